# baseline (speedup 1.0000x reference)
import functools

import jax
import jax.numpy as jnp
from jax import lax
from jax.experimental import pallas as pl
from jax.experimental.pallas import tpu as pltpu
from jax.experimental.pallas import tpu_sc as plsc

VOCAB = 100000
MAXLEN = 200
EMBED = 128
BATCH = 1024
EPS = 1e-05

NC = 2
NS = 16
NW = NC * NS
ROWS_PER_W = BATCH // NW
NB = 3
STEPS = 10
NV = EMBED // 16

UNROLL = 8

SUBS = ((0, 56), (56, 48), (104, 48), (152, 48))


def _rsqrt(v):
    i = lax.bitcast_convert_type(v, jnp.int32)
    i = jnp.int32(0x5F3759DF) - (i >> 1)
    y = lax.bitcast_convert_type(i, jnp.float32)
    for _ in range(2):
        y = y * (1.5 - 0.5 * v * y * y)
    return y


def _ln_range(rows_v, pos_v, gs, bs, lo, n):
    def one_token(i):
        xs = [rows_v[i, pl.ds(k * 16, 16)] + pos_v[i, pl.ds(k * 16, 16)]
              for k in range(NV)]
        s = xs[0]
        sq = xs[0] * xs[0]
        for k in range(1, NV):
            s = s + xs[k]
            sq = sq + xs[k] * xs[k]
        ssum = plsc.cumsum(s)[15]
        sqsum = plsc.cumsum(sq)[15]
        mean = ssum * (1.0 / EMBED)
        var = sqsum * (1.0 / EMBED) - mean * mean
        rstd = _rsqrt(var + EPS)
        shift = -mean * rstd
        for k in range(NV):
            t = xs[k] * rstd + shift
            rows_v[i, pl.ds(k * 16, 16)] = t * gs[k] + bs[k]

    def body(ii, carry):
        for u in range(UNROLL):
            one_token(lo + ii * UNROLL + u)
        return carry

    lax.fori_loop(0, n // UNROLL, body, 0)


def _body(x_hbm, wtab_hbm, pos_hbm, g_hbm, b_hbm, out_hbm,
          idx0, idx1, idx2, rows0, rows1, rows2, pos_v, g_v, b_v,
          is0, is1, is2, gs0, gs1, gs2, ss0, ss1, ss2):
    wid = lax.axis_index("s") * NC + lax.axis_index("c")
    r0 = wid * ROWS_PER_W
    pltpu.sync_copy(pos_hbm, pos_v)
    pltpu.sync_copy(g_hbm, g_v)
    pltpu.sync_copy(b_hbm, b_v)
    gs = [g_v[pl.ds(k * 16, 16)] for k in range(NV)]
    bs = [b_v[pl.ds(k * 16, 16)] for k in range(NV)]

    idxs = (idx0, idx1, idx2)
    rows = (rows0, rows1, rows2)
    isems = (is0, is1, is2)
    gsems = (gs0, gs1, gs2)
    ssems = (ss0, ss1, ss2)

    def fire_gathers(b):
        for lo, n in SUBS:
            pltpu.async_copy(wtab_hbm.at[idxs[b].at[pl.ds(lo, n)]],
                             rows[b].at[pl.ds(lo, n)], gsems[b])

    def wait_gather_sub(b, q):
        lo, n = SUBS[q]
        pltpu.make_async_copy(wtab_hbm.at[idxs[b].at[pl.ds(lo, n)]],
                              rows[b].at[pl.ds(lo, n)], gsems[b]).wait()

    def fire_store_sub(b, r, q):
        lo, n = SUBS[q]
        pltpu.async_copy(rows[b].at[pl.ds(lo, n)],
                         out_hbm.at[r, pl.ds(lo, n)], ssems[b])

    def drain_stores(b, r):
        for lo, n in SUBS:
            pltpu.make_async_copy(rows[b].at[pl.ds(lo, n)],
                                  out_hbm.at[r, pl.ds(lo, n)], ssems[b]).wait()

    def wait_idx(b):
        pltpu.make_async_copy(x_hbm.at[r0], idxs[b], isems[b]).wait()

    def process(b, r, idx_prefetch=None):
        for q in range(len(SUBS)):
            wait_gather_sub(b, q)
            if q == len(SUBS) - 1 and idx_prefetch is not None:
                idx_prefetch()
            lo, n = SUBS[q]
            _ln_range(rows[b], pos_v, gs, bs, lo, n)
            fire_store_sub(b, r, q)

    pltpu.async_copy(x_hbm.at[r0], idx0, is0)
    pltpu.async_copy(x_hbm.at[r0 + 1], idx1, is1)
    pltpu.async_copy(x_hbm.at[r0 + 2], idx2, is2)
    wait_idx(0)
    fire_gathers(0)
    wait_idx(1)
    fire_gathers(1)

    def body(i, carry):
        for u in range(NB):
            b = u
            c = r0 + 3 * i + u

            def prefetch(b=b, c=c, u=u):
                if u < 2:
                    pltpu.async_copy(x_hbm.at[c + 3], idxs[b], isems[b])
                else:
                    @pl.when(i < STEPS - 1)
                    def _():
                        pltpu.async_copy(x_hbm.at[c + 3], idxs[b], isems[b])

            process(b, c, prefetch)

            nb = (u + 2) % NB
            if u == 0:
                @pl.when(i > 0)
                def _():
                    drain_stores(nb, c - 1)
            else:
                drain_stores(nb, c - 1)
            wait_idx(nb)
            fire_gathers(nb)
        return carry

    lax.fori_loop(0, STEPS, body, 0)
    process(0, r0 + 30)
    process(1, r0 + 31)
    drain_stores(2, r0 + 29)
    drain_stores(0, r0 + 30)
    drain_stores(1, r0 + 31)


def kernel(x, word_embeddings, pos_embeddings, gamma, beta):
    mesh = plsc.VectorSubcoreMesh(core_axis_name="c", subcore_axis_name="s",
                                  num_cores=NC, num_subcores=NS)
    f = pl.kernel(
        _body,
        out_type=jax.ShapeDtypeStruct((BATCH, MAXLEN, EMBED), jnp.float32),
        mesh=mesh,
        compiler_params=pltpu.CompilerParams(needs_layout_passes=False),
        scratch_types=[
            pltpu.VMEM((MAXLEN,), jnp.int32),
            pltpu.VMEM((MAXLEN,), jnp.int32),
            pltpu.VMEM((MAXLEN,), jnp.int32),
            pltpu.VMEM((MAXLEN, EMBED), jnp.float32),
            pltpu.VMEM((MAXLEN, EMBED), jnp.float32),
            pltpu.VMEM((MAXLEN, EMBED), jnp.float32),
            pltpu.VMEM((MAXLEN, EMBED), jnp.float32),
            pltpu.VMEM((EMBED,), jnp.float32),
            pltpu.VMEM((EMBED,), jnp.float32),
            pltpu.SemaphoreType.DMA,
            pltpu.SemaphoreType.DMA,
            pltpu.SemaphoreType.DMA,
            pltpu.SemaphoreType.DMA,
            pltpu.SemaphoreType.DMA,
            pltpu.SemaphoreType.DMA,
            pltpu.SemaphoreType.DMA,
            pltpu.SemaphoreType.DMA,
            pltpu.SemaphoreType.DMA,
        ],
    )
    return f(x, word_embeddings, pos_embeddings, gamma, beta)


# speedup vs baseline: 1.0891x; 1.0891x over previous
import functools

import jax
import jax.numpy as jnp
from jax import lax
from jax.experimental import pallas as pl
from jax.experimental.pallas import tpu as pltpu
from jax.experimental.pallas import tpu_sc as plsc

VOCAB = 100000
MAXLEN = 200
EMBED = 128
BATCH = 1024
EPS = 1e-05

NC = 2
NS = 16
NW = NC * NS
ROWS_PER_W = BATCH // NW
HALF = ROWS_PER_W // 2
NV = EMBED // 16

UNROLL = 8

SUBS = ((0, 56), (56, 48), (104, 48), (152, 48))


def _rsqrt(v):
    i = lax.bitcast_convert_type(v, jnp.int32)
    i = jnp.int32(0x5F3759DF) - (i >> 1)
    y = lax.bitcast_convert_type(i, jnp.float32)
    for _ in range(2):
        y = y * (1.5 - 0.5 * v * y * y)
    return y


def _ln_range(rows_v, pos_v, gs, bs, lo, n):
    def one_token(i):
        xs = [rows_v[i, pl.ds(k * 16, 16)] + pos_v[i, pl.ds(k * 16, 16)]
              for k in range(NV)]
        s = xs[0]
        sq = xs[0] * xs[0]
        for k in range(1, NV):
            s = s + xs[k]
            sq = sq + xs[k] * xs[k]
        ssum = plsc.cumsum(s)[15]
        sqsum = plsc.cumsum(sq)[15]
        mean = ssum * (1.0 / EMBED)
        var = sqsum * (1.0 / EMBED) - mean * mean
        rstd = _rsqrt(var + EPS)
        shift = -mean * rstd
        for k in range(NV):
            t = xs[k] * rstd + shift
            rows_v[i, pl.ds(k * 16, 16)] = t * gs[k] + bs[k]

    def body(ii, carry):
        for u in range(UNROLL):
            one_token(lo + ii * UNROLL + u)
        return carry

    lax.fori_loop(0, n // UNROLL, body, 0)


def _body(x_hbm, wtab_hbm, pos_hbm, g_hbm, b_hbm, out_hbm,
          idx_a, idx_b, rows_a, rows_b, pos_v, g_v, b_v,
          isa, isb, gsa, gsb, ssa, ssb):
    wid = lax.axis_index("s") * NC + lax.axis_index("c")
    r0 = wid * ROWS_PER_W
    pltpu.sync_copy(pos_hbm, pos_v)
    pltpu.sync_copy(g_hbm, g_v)
    pltpu.sync_copy(b_hbm, b_v)
    gs = [g_v[pl.ds(k * 16, 16)] for k in range(NV)]
    bs = [b_v[pl.ds(k * 16, 16)] for k in range(NV)]

    def fire_gathers(idx_v, rows_v, sem):
        for lo, n in SUBS:
            pltpu.async_copy(wtab_hbm.at[idx_v.at[pl.ds(lo, n)]],
                             rows_v.at[pl.ds(lo, n)], sem)

    def wait_gather_sub(idx_v, rows_v, sem, q):
        lo, n = SUBS[q]
        pltpu.make_async_copy(wtab_hbm.at[idx_v.at[pl.ds(lo, n)]],
                              rows_v.at[pl.ds(lo, n)], sem).wait()

    def fire_store_sub(rows_v, r, sem, q):
        lo, n = SUBS[q]
        pltpu.async_copy(rows_v.at[pl.ds(lo, n)],
                         out_hbm.at[r, pl.ds(lo, n)], sem)

    def drain_stores(rows_v, r, sem):
        for lo, n in SUBS:
            pltpu.make_async_copy(rows_v.at[pl.ds(lo, n)],
                                  out_hbm.at[r, pl.ds(lo, n)], sem).wait()

    def wait_idx(idx_v, sem):
        pltpu.make_async_copy(x_hbm.at[r0], idx_v, sem).wait()

    def process(idx_v, rows_v, r, gsem, ssem, idx_prefetch):
        for q in range(len(SUBS)):
            wait_gather_sub(idx_v, rows_v, gsem, q)
            if q == len(SUBS) - 1:
                idx_prefetch()
            lo, n = SUBS[q]
            _ln_range(rows_v, pos_v, gs, bs, lo, n)
            fire_store_sub(rows_v, r, ssem, q)

    pltpu.async_copy(x_hbm.at[r0], idx_a, isa)
    wait_idx(idx_a, isa)
    fire_gathers(idx_a, rows_a, gsa)
    pltpu.async_copy(x_hbm.at[r0 + 1], idx_b, isb)

    def body(i, carry):
        ca = r0 + 2 * i
        cb = ca + 1
        wait_idx(idx_b, isb)

        @pl.when(i > 0)
        def _():
            drain_stores(rows_b, cb - 2, ssb)

        fire_gathers(idx_b, rows_b, gsb)

        def prefetch_a():
            @pl.when(i < HALF - 1)
            def _():
                pltpu.async_copy(x_hbm.at[ca + 2], idx_a, isa)

        process(idx_a, rows_a, ca, gsa, ssa, prefetch_a)

        @pl.when(i < HALF - 1)
        def _():
            wait_idx(idx_a, isa)
            drain_stores(rows_a, ca, ssa)
            fire_gathers(idx_a, rows_a, gsa)

        def prefetch_b():
            @pl.when(i < HALF - 1)
            def _():
                pltpu.async_copy(x_hbm.at[cb + 2], idx_b, isb)

        process(idx_b, rows_b, cb, gsb, ssb, prefetch_b)
        return carry

    lax.fori_loop(0, HALF, body, 0)
    drain_stores(rows_a, r0 + ROWS_PER_W - 2, ssa)
    drain_stores(rows_b, r0 + ROWS_PER_W - 1, ssb)


def kernel(x, word_embeddings, pos_embeddings, gamma, beta):
    mesh = plsc.VectorSubcoreMesh(core_axis_name="c", subcore_axis_name="s",
                                  num_cores=NC, num_subcores=NS)
    f = pl.kernel(
        _body,
        out_type=jax.ShapeDtypeStruct((BATCH, MAXLEN, EMBED), jnp.float32),
        mesh=mesh,
        compiler_params=pltpu.CompilerParams(needs_layout_passes=False),
        scratch_types=[
            pltpu.VMEM((MAXLEN,), jnp.int32),
            pltpu.VMEM((MAXLEN,), jnp.int32),
            pltpu.VMEM((MAXLEN, EMBED), jnp.float32),
            pltpu.VMEM((MAXLEN, EMBED), jnp.float32),
            pltpu.VMEM((MAXLEN, EMBED), jnp.float32),
            pltpu.VMEM((EMBED,), jnp.float32),
            pltpu.VMEM((EMBED,), jnp.float32),
            pltpu.SemaphoreType.DMA,
            pltpu.SemaphoreType.DMA,
            pltpu.SemaphoreType.DMA,
            pltpu.SemaphoreType.DMA,
            pltpu.SemaphoreType.DMA,
            pltpu.SemaphoreType.DMA,
        ],
    )
    return f(x, word_embeddings, pos_embeddings, gamma, beta)
